# baseline (speedup 1.0000x reference)
import functools

import jax
import jax.numpy as jnp
from jax import lax
from jax.experimental import pallas as pl
from jax.experimental.pallas import tpu as pltpu
from jax.experimental.pallas import tpu_sc as plsc

SEP = 29871
INV_TEMP = 1.0 / 0.7
B = 8
LQ = 512
LR = 256
VOCAB = 32000
D = 1024

NC, NS = 2, 16
NW = NC * NS
TPW = (B * LR) // NW

RB = 1024
NRB = (B * LR) // RB
VB = 3200
NVB = VOCAB // VB


def _sc_select_and_gather(queries, responses, emb_table):
    mesh = plsc.VectorSubcoreMesh(
        core_axis_name="c", subcore_axis_name="s", num_cores=NC, num_subcores=NS
    )

    @functools.partial(
        pl.kernel,
        out_type=(
            jax.ShapeDtypeStruct((B * LR, D), jnp.float32),
            jax.ShapeDtypeStruct((B * LR,), jnp.int32),
        ),
        mesh=mesh,
        compiler_params=pltpu.CompilerParams(needs_layout_passes=False),
        scratch_types=[
            pltpu.VMEM((LQ,), jnp.int32),
            pltpu.VMEM((LR,), jnp.int32),
            pltpu.VMEM((TPW,), jnp.int32),
            pltpu.VMEM((TPW, D), jnp.float32),
            pltpu.SemaphoreType.DMA,
        ],
    )
    def sck(q_hbm, r_hbm, emb_hbm, e_out, tok_out, qrow, rrow, idxb, rows, sem):
        wid = lax.axis_index("s") * NC + lax.axis_index("c")
        b = wid // (NW // B)
        t0 = (wid % (NW // B)) * TPW
        base = b * LR + t0
        pltpu.sync_copy(q_hbm.at[b], qrow)
        pltpu.sync_copy(r_hbm.at[b], rrow)

        io16 = lax.iota(jnp.int32, 16)

        def body(k, carry):
            cnt, idx3 = carry
            v = qrow[pl.ds(k * 16, 16)]
            mb = v == SEP
            n = plsc.all_reduce_population_count(mb)
            f0 = plsc.all_reduce_ffs(mb)
            mb2 = mb & (io16 > f0)
            f1 = plsc.all_reduce_ffs(mb2)
            mb3 = mb2 & (io16 > f1)
            f2 = plsc.all_reduce_ffs(mb3)
            r = 3 - cnt
            lane = jnp.where(r == 1, f0, jnp.where(r == 2, f1, f2))
            hit = (cnt < 3) & (cnt + n >= 3)
            idx3 = jnp.where(hit, 16 * k + lane, idx3)
            return cnt + n, idx3

        _, idx3 = lax.fori_loop(
            0,
            LQ // 16,
            body,
            (jnp.zeros((16,), jnp.int32), jnp.zeros((16,), jnp.int32)),
        )

        qtok = plsc.load_gather(qrow, [idx3 - 1])
        for j in range(TPW // 16):
            li = lax.iota(jnp.int32, 16) + (t0 - 1 + 16 * j)
            g = plsc.load_gather(rrow, [jnp.maximum(li, 0)])
            idxb[pl.ds(j * 16, 16)] = jnp.where(li < 0, qtok, g)

        pltpu.async_copy(emb_hbm.at[idxb], rows, sem).wait()
        pltpu.sync_copy(rows, e_out.at[pl.ds(base, TPW)])
        pltpu.sync_copy(idxb, tok_out.at[pl.ds(base, TPW)])

    return sck(queries, responses, emb_table)


def _hidden_body(e_ref, tok_ref, w1_ref, h_ref):
    e = e_ref[...]
    msk = (tok_ref[0, 0, :] != 0).astype(jnp.float32)
    e = e * msk[:, None]
    h = jnp.tanh(
        jnp.dot(e.astype(jnp.bfloat16), w1_ref[...], preferred_element_type=jnp.float32)
    )
    h_ref[...] = (h * INV_TEMP).astype(jnp.bfloat16)


def _tc_hidden(E, tok, w1_bf):
    return pl.pallas_call(
        _hidden_body,
        grid=(NRB,),
        in_specs=[
            pl.BlockSpec((RB, D), lambda r: (r, 0)),
            pl.BlockSpec((1, 1, RB), lambda r: (r, 0, 0)),
            pl.BlockSpec((D, D), lambda r: (0, 0)),
        ],
        out_specs=pl.BlockSpec((RB, D), lambda r: (r, 0)),
        out_shape=jax.ShapeDtypeStruct((B * LR, D), jnp.bfloat16),
    )(E, tok.reshape(NRB, 1, RB), w1_bf)


VC = 640
NVC = VB // VC


def _lse_body(h_ref, w_ref, cols_ref, resp_ref, aqm_ref, out_ref, s_ref, t_ref):
    vb = pl.program_id(0)
    rb = pl.program_id(1)
    h = h_ref[pl.ds(rb * RB, RB), :]
    rcol = resp_ref[0, 0, :][:, None]
    s_loc = jnp.zeros((RB, 128), jnp.float32)
    t_loc = jnp.zeros((RB, 128), jnp.float32)
    w_bfs = [
        w_ref[:, c * VC:(c + 1) * VC].astype(jnp.bfloat16) for c in range(NVC)
    ]
    for c in range(NVC):
        l = jnp.dot(h, w_bfs[c], preferred_element_type=jnp.float32)
        cols = cols_ref[0, c * VC:(c + 1) * VC][None, :]
        e = jnp.exp(l)
        m = jnp.where(cols == rcol, l, 0.0)
        for g in range(VC // 128):
            s_loc = s_loc + e[:, g * 128:(g + 1) * 128]
            t_loc = t_loc + m[:, g * 128:(g + 1) * 128]

    @pl.when(vb == 0)
    def _():
        s_ref[rb] = s_loc
        t_ref[rb] = t_loc

    @pl.when(vb > 0)
    def _():
        s_ref[rb] = s_ref[rb] + s_loc
        t_ref[rb] = t_ref[rb] + t_loc

    @pl.when(vb == NVB - 1)
    def _():
        out_ref[0, 0, :] = (
            jnp.sum(t_ref[rb], axis=1) - jnp.log(jnp.sum(s_ref[rb], axis=1))
        ) * aqm_ref[0, 0, :]


def _tc_lse(H, wlm, resp, aqm):
    cols = jnp.arange(VOCAB, dtype=jnp.int32).reshape(1, VOCAB)
    out = pl.pallas_call(
        _lse_body,
        grid=(NVB, NRB),
        in_specs=[
            pl.BlockSpec((B * LR, D), lambda v, r: (0, 0)),
            pl.BlockSpec((D, VB), lambda v, r: (0, v)),
            pl.BlockSpec((1, VB), lambda v, r: (0, v)),
            pl.BlockSpec((1, 1, RB), lambda v, r: (r, 0, 0)),
            pl.BlockSpec((1, 1, RB), lambda v, r: (r, 0, 0)),
        ],
        out_specs=pl.BlockSpec((1, 1, RB), lambda v, r: (r, 0, 0)),
        out_shape=jax.ShapeDtypeStruct((NRB, 1, RB), jnp.float32),
        scratch_shapes=[
            pltpu.VMEM((NRB, RB, 128), jnp.float32),
            pltpu.VMEM((NRB, RB, 128), jnp.float32),
        ],
    )(H, wlm, cols, resp.reshape(NRB, 1, RB), aqm.reshape(NRB, 1, RB))
    return out.reshape(B, LR)


def kernel(queries, query_attn_masks, responses, AnswerQuestionMASK, images, emb_table, W1, W_lm):
    E, tok = _sc_select_and_gather(queries, responses, emb_table)
    w1_bf = W1.astype(jnp.bfloat16)
    H = _tc_hidden(E, tok, w1_bf)
    return _tc_lse(H, W_lm, responses, AnswerQuestionMASK)


# speedup vs baseline: 1.0132x; 1.0132x over previous
import functools

import jax
import jax.numpy as jnp
from jax import lax
from jax.experimental import pallas as pl
from jax.experimental.pallas import tpu as pltpu
from jax.experimental.pallas import tpu_sc as plsc

SEP = 29871
INV_TEMP = 1.0 / 0.7
B = 8
LQ = 512
LR = 256
VOCAB = 32000
D = 1024

NC, NS = 2, 16
NW = NC * NS
TPW = (B * LR) // NW

RB = 1024
NRB = (B * LR) // RB
VB = 3200
NVB = VOCAB // VB


def _sc_select_and_gather(queries, responses, emb_table):
    mesh = plsc.VectorSubcoreMesh(
        core_axis_name="c", subcore_axis_name="s", num_cores=NC, num_subcores=NS
    )

    @functools.partial(
        pl.kernel,
        out_type=(
            jax.ShapeDtypeStruct((B * LR, D), jnp.float32),
            jax.ShapeDtypeStruct((B * LR,), jnp.int32),
        ),
        mesh=mesh,
        compiler_params=pltpu.CompilerParams(needs_layout_passes=False),
        scratch_types=[
            pltpu.VMEM((LQ,), jnp.int32),
            pltpu.VMEM((LR,), jnp.int32),
            pltpu.VMEM((TPW,), jnp.int32),
            pltpu.VMEM((TPW, D), jnp.float32),
            pltpu.SemaphoreType.DMA,
        ],
    )
    def sck(q_hbm, r_hbm, emb_hbm, e_out, tok_out, qrow, rrow, idxb, rows, sem):
        wid = lax.axis_index("s") * NC + lax.axis_index("c")
        b = wid // (NW // B)
        t0 = (wid % (NW // B)) * TPW
        base = b * LR + t0
        pltpu.sync_copy(q_hbm.at[b], qrow)
        pltpu.sync_copy(r_hbm.at[b], rrow)

        io16 = lax.iota(jnp.int32, 16)

        def body(k, carry):
            cnt, idx3 = carry
            v = qrow[pl.ds(k * 16, 16)]
            mb = v == SEP
            n = plsc.all_reduce_population_count(mb)
            f0 = plsc.all_reduce_ffs(mb)
            mb2 = mb & (io16 > f0)
            f1 = plsc.all_reduce_ffs(mb2)
            mb3 = mb2 & (io16 > f1)
            f2 = plsc.all_reduce_ffs(mb3)
            r = 3 - cnt
            lane = jnp.where(r == 1, f0, jnp.where(r == 2, f1, f2))
            hit = (cnt < 3) & (cnt + n >= 3)
            idx3 = jnp.where(hit, 16 * k + lane, idx3)
            return cnt + n, idx3

        _, idx3 = lax.fori_loop(
            0,
            LQ // 16,
            body,
            (jnp.zeros((16,), jnp.int32), jnp.zeros((16,), jnp.int32)),
        )

        qtok = plsc.load_gather(qrow, [idx3 - 1])
        for j in range(TPW // 16):
            li = lax.iota(jnp.int32, 16) + (t0 - 1 + 16 * j)
            g = plsc.load_gather(rrow, [jnp.maximum(li, 0)])
            idxb[pl.ds(j * 16, 16)] = jnp.where(li < 0, qtok, g)

        pltpu.async_copy(emb_hbm.at[idxb], rows, sem).wait()
        pltpu.sync_copy(rows, e_out.at[pl.ds(base, TPW)])
        pltpu.sync_copy(idxb, tok_out.at[pl.ds(base, TPW)])

    return sck(queries, responses, emb_table)


VC = 640
NVC = VB // VC


def _fused_body(e_ref, tok_ref, w1_ref, w_ref, cols_ref, resp_ref, aqm_ref,
                out_ref, hs_ref, s_ref, t_ref):
    vb = pl.program_id(0)
    rb = pl.program_id(1)

    @pl.when(vb == 0)
    def _():
        e = e_ref[...]
        msk = (tok_ref[0, 0, :] != 0).astype(jnp.float32)
        e = e * msk[:, None]
        hid = jnp.tanh(
            jnp.dot(
                e.astype(jnp.bfloat16),
                w1_ref[...].astype(jnp.bfloat16),
                preferred_element_type=jnp.float32,
            )
        )
        hs_ref[pl.ds(rb * RB, RB), :] = (hid * INV_TEMP).astype(jnp.bfloat16)

    @pl.when(vb > 0)
    def _():
        h = hs_ref[pl.ds(rb * RB, RB), :]
        rcol = resp_ref[0, 0, :][:, None]
        s_loc = jnp.zeros((RB, 128), jnp.float32)
        t_loc = jnp.zeros((RB, 128), jnp.float32)
        for c in range(NVC):
            w_bf = w_ref[:, c * VC:(c + 1) * VC].astype(jnp.bfloat16)
            l = jnp.dot(h, w_bf, preferred_element_type=jnp.float32)
            cols = cols_ref[0, c * VC:(c + 1) * VC][None, :]
            e = jnp.exp(l)
            m = jnp.where(cols == rcol, l, 0.0)
            for g in range(VC // 128):
                s_loc = s_loc + e[:, g * 128:(g + 1) * 128]
                t_loc = t_loc + m[:, g * 128:(g + 1) * 128]

        @pl.when(vb == 1)
        def _():
            s_ref[rb] = s_loc
            t_ref[rb] = t_loc

        @pl.when(vb > 1)
        def _():
            s_ref[rb] = s_ref[rb] + s_loc
            t_ref[rb] = t_ref[rb] + t_loc

        @pl.when(vb == NVB)
        def _():
            out_ref[0, 0, :] = (
                jnp.sum(t_ref[rb], axis=1) - jnp.log(jnp.sum(s_ref[rb], axis=1))
            ) * aqm_ref[0, 0, :]


def _tc_fused(E, tok, W1, wlm, resp, aqm):
    cols = jnp.arange(VOCAB, dtype=jnp.int32).reshape(1, VOCAB)
    out = pl.pallas_call(
        _fused_body,
        grid=(NVB + 1, NRB),
        in_specs=[
            pl.BlockSpec((RB, D), lambda v, r: (jnp.where(v == 0, r, 0), 0)),
            pl.BlockSpec((1, 1, RB), lambda v, r: (jnp.where(v == 0, r, 0), 0, 0)),
            pl.BlockSpec((D, D), lambda v, r: (0, 0)),
            pl.BlockSpec((D, VB), lambda v, r: (0, jnp.maximum(v - 1, 0))),
            pl.BlockSpec((1, VB), lambda v, r: (0, jnp.maximum(v - 1, 0))),
            pl.BlockSpec((1, 1, RB), lambda v, r: (r, 0, 0)),
            pl.BlockSpec((1, 1, RB), lambda v, r: (r, 0, 0)),
        ],
        out_specs=pl.BlockSpec((1, 1, RB), lambda v, r: (r, 0, 0)),
        out_shape=jax.ShapeDtypeStruct((NRB, 1, RB), jnp.float32),
        scratch_shapes=[
            pltpu.VMEM((B * LR, D), jnp.bfloat16),
            pltpu.VMEM((NRB, RB, 128), jnp.float32),
            pltpu.VMEM((NRB, RB, 128), jnp.float32),
        ],
    )(E, tok.reshape(NRB, 1, RB), W1, wlm, cols,
      resp.reshape(NRB, 1, RB), aqm.reshape(NRB, 1, RB))
    return out.reshape(B, LR)


def kernel(queries, query_attn_masks, responses, AnswerQuestionMASK, images, emb_table, W1, W_lm):
    E, tok = _sc_select_and_gather(queries, responses, emb_table)
    return _tc_fused(E, tok, W1, W_lm, responses, AnswerQuestionMASK)


# speedup vs baseline: 1.0225x; 1.0091x over previous
import functools

import jax
import jax.numpy as jnp
from jax import lax
from jax.experimental import pallas as pl
from jax.experimental.pallas import tpu as pltpu
from jax.experimental.pallas import tpu_sc as plsc

SEP = 29871
INV_TEMP = 1.0 / 0.7
B = 8
LQ = 512
LR = 256
VOCAB = 32000
D = 1024

NC, NS = 2, 16
NW = NC * NS
TPW = (B * LR) // NW

RB = 1024
NRB = (B * LR) // RB
VB = 3200
NVB = VOCAB // VB


def _sc_select_and_gather(queries, responses, emb_table):
    mesh = plsc.VectorSubcoreMesh(
        core_axis_name="c", subcore_axis_name="s", num_cores=NC, num_subcores=NS
    )

    @functools.partial(
        pl.kernel,
        out_type=(
            jax.ShapeDtypeStruct((B * LR, D), jnp.float32),
            jax.ShapeDtypeStruct((B * LR,), jnp.int32),
        ),
        mesh=mesh,
        compiler_params=pltpu.CompilerParams(needs_layout_passes=False),
        scratch_types=[
            pltpu.VMEM((LQ,), jnp.int32),
            pltpu.VMEM((LR,), jnp.int32),
            pltpu.VMEM((TPW,), jnp.int32),
            pltpu.VMEM((TPW // 2, D), jnp.float32),
            pltpu.VMEM((TPW // 2, D), jnp.float32),
            pltpu.SemaphoreType.DMA,
            pltpu.SemaphoreType.DMA,
            pltpu.SemaphoreType.DMA,
        ],
    )
    def sck(q_hbm, r_hbm, emb_hbm, e_out, tok_out, qrow, rrow, idxb,
            rows0, rows1, sem0, sem1, semw):
        wid = lax.axis_index("s") * NC + lax.axis_index("c")
        b = wid // (NW // B)
        t0 = (wid % (NW // B)) * TPW
        base = b * LR + t0
        pltpu.sync_copy(r_hbm.at[b], rrow)
        io16 = lax.iota(jnp.int32, 16)

        @pl.when(t0 == 0)
        def _():
            pltpu.sync_copy(q_hbm.at[b], qrow)

            def body(k, carry):
                cnt, idx3 = carry
                v = qrow[pl.ds(k * 16, 16)]
                mb = v == SEP
                n = plsc.all_reduce_population_count(mb)
                f0 = plsc.all_reduce_ffs(mb)
                mb2 = mb & (io16 > f0)
                f1 = plsc.all_reduce_ffs(mb2)
                mb3 = mb2 & (io16 > f1)
                f2 = plsc.all_reduce_ffs(mb3)
                r = 3 - cnt
                lane = jnp.where(r == 1, f0, jnp.where(r == 2, f1, f2))
                hit = (cnt < 3) & (cnt + n >= 3)
                idx3 = jnp.where(hit, 16 * k + lane, idx3)
                return cnt + n, idx3

            _, idx3 = lax.fori_loop(
                0,
                LQ // 16,
                body,
                (jnp.zeros((16,), jnp.int32), jnp.zeros((16,), jnp.int32)),
            )
            qtok = plsc.load_gather(qrow, [idx3 - 1])
            li = io16 - 1
            g = plsc.load_gather(rrow, [jnp.maximum(li, 0)])
            idxb[pl.ds(0, 16)] = jnp.where(li < 0, qtok, g)

        @pl.when(t0 > 0)
        def _():
            idxb[pl.ds(0, 16)] = plsc.load_gather(rrow, [io16 + (t0 - 1)])

        for j in range(1, TPW // 16):
            idxb[pl.ds(j * 16, 16)] = plsc.load_gather(
                rrow, [io16 + (t0 - 1 + 16 * j)]
            )

        g0 = pltpu.async_copy(emb_hbm.at[idxb.at[pl.ds(0, TPW // 2)]], rows0, sem0)
        g1 = pltpu.async_copy(emb_hbm.at[idxb.at[pl.ds(TPW // 2, TPW // 2)]], rows1, sem1)
        g0.wait()
        w0 = pltpu.async_copy(rows0, e_out.at[pl.ds(base, TPW // 2)], semw)
        g1.wait()
        w1 = pltpu.async_copy(rows1, e_out.at[pl.ds(base + TPW // 2, TPW // 2)], semw)
        pltpu.sync_copy(idxb, tok_out.at[pl.ds(base, TPW)])
        w0.wait()
        w1.wait()

    return sck(queries, responses, emb_table)


VC = 640
NVC = VB // VC


def _fused_body(e_ref, tok_ref, w1_ref, w_ref, cols_ref, resp_ref, aqm_ref,
                out_ref, hs_ref, s_ref, t_ref):
    vb = pl.program_id(0)
    rb = pl.program_id(1)

    @pl.when(vb == 0)
    def _():
        e = e_ref[...]
        msk = (tok_ref[0, 0, :] != 0).astype(jnp.float32)
        e = e * msk[:, None]
        hid = jnp.tanh(
            jnp.dot(
                e.astype(jnp.bfloat16),
                w1_ref[...].astype(jnp.bfloat16),
                preferred_element_type=jnp.float32,
            )
        )
        hs_ref[pl.ds(rb * RB, RB), :] = (hid * INV_TEMP).astype(jnp.bfloat16)

    @pl.when(vb > 0)
    def _():
        h = hs_ref[pl.ds(rb * RB, RB), :]
        rcol = resp_ref[0, 0, :][:, None]
        s_loc = jnp.zeros((RB, 128), jnp.float32)
        t_loc = jnp.zeros((RB, 128), jnp.float32)
        for c in range(NVC):
            w_bf = w_ref[:, c * VC:(c + 1) * VC].astype(jnp.bfloat16)
            l = jnp.dot(h, w_bf, preferred_element_type=jnp.float32)
            cols = cols_ref[0, c * VC:(c + 1) * VC][None, :]
            e = jnp.exp(l)
            m = jnp.where(cols == rcol, e, 0.0)
            for g in range(VC // 128):
                s_loc = s_loc + e[:, g * 128:(g + 1) * 128]
                t_loc = t_loc + m[:, g * 128:(g + 1) * 128]

        @pl.when(vb == 1)
        def _():
            s_ref[rb] = s_loc
            t_ref[rb] = t_loc

        @pl.when(vb > 1)
        def _():
            s_ref[rb] = s_ref[rb] + s_loc
            t_ref[rb] = t_ref[rb] + t_loc

        @pl.when(vb == NVB)
        def _():
            out_ref[0, 0, :] = (
                jnp.log(jnp.sum(t_ref[rb], axis=1))
                - jnp.log(jnp.sum(s_ref[rb], axis=1))
            ) * aqm_ref[0, 0, :]


def _tc_fused(E, tok, W1, wlm, resp, aqm):
    cols = jnp.arange(VOCAB, dtype=jnp.int32).reshape(1, VOCAB)
    out = pl.pallas_call(
        _fused_body,
        grid=(NVB + 1, NRB),
        in_specs=[
            pl.BlockSpec((RB, D), lambda v, r: (jnp.where(v == 0, r, 0), 0)),
            pl.BlockSpec((1, 1, RB), lambda v, r: (jnp.where(v == 0, r, 0), 0, 0)),
            pl.BlockSpec((D, D), lambda v, r: (0, 0)),
            pl.BlockSpec((D, VB), lambda v, r: (0, jnp.maximum(v - 1, 0))),
            pl.BlockSpec((1, VB), lambda v, r: (0, jnp.maximum(v - 1, 0))),
            pl.BlockSpec((1, 1, RB), lambda v, r: (r, 0, 0)),
            pl.BlockSpec((1, 1, RB), lambda v, r: (r, 0, 0)),
        ],
        out_specs=pl.BlockSpec((1, 1, RB), lambda v, r: (r, 0, 0)),
        out_shape=jax.ShapeDtypeStruct((NRB, 1, RB), jnp.float32),
        scratch_shapes=[
            pltpu.VMEM((B * LR, D), jnp.bfloat16),
            pltpu.VMEM((NRB, RB, 128), jnp.float32),
            pltpu.VMEM((NRB, RB, 128), jnp.float32),
        ],
    )(E, tok.reshape(NRB, 1, RB), W1, wlm, cols,
      resp.reshape(NRB, 1, RB), aqm.reshape(NRB, 1, RB))
    return out.reshape(B, LR)


def kernel(queries, query_attn_masks, responses, AnswerQuestionMASK, images, emb_table, W1, W_lm):
    E, tok = _sc_select_and_gather(queries, responses, emb_table)
    return _tc_fused(E, tok, W1, W_lm, responses, AnswerQuestionMASK)
